# Initial kernel scaffold; baseline (speedup 1.0000x reference)
import jax
import jax.numpy as jnp
from jax.experimental import pallas as pl


def kernel(data, W, att_src, att_dst, bias):
    raise NotImplementedError("write your pallas kernel here")


# speedup vs baseline: 3207.5011x; 3207.5011x over previous
import jax
import jax.numpy as jnp
from jax.experimental import pallas as pl

N = 1024
OUT_CH = 128


def _gat_dense_kernel(data_ref, w_ref, asrc_ref, adst_ref, bias_ref, out_ref):
    xp = jnp.dot(data_ref[:], w_ref[:], preferred_element_type=jnp.float32)
    a_s = jax.lax.dot_general(
        asrc_ref[:], xp, (((1,), (1,)), ((), ())),
        preferred_element_type=jnp.float32)
    a_d = jnp.dot(xp, adst_ref[:], preferred_element_type=jnp.float32)
    e = a_d + a_s
    e = jnp.where(e > 0, e, 0.2 * e)
    row = jax.lax.broadcasted_iota(jnp.int32, (N, N), 0)
    col = jax.lax.broadcasted_iota(jnp.int32, (N, N), 1)
    mask = col <= row
    e = jnp.where(mask, e, -1e30)
    m = jnp.max(e, axis=1, keepdims=True)
    ex = jnp.where(mask, jnp.exp(e - m), 0.0)
    denom = jnp.sum(ex, axis=1, keepdims=True)
    alpha = ex / denom
    out = jnp.dot(alpha, xp, preferred_element_type=jnp.float32) + bias_ref[:]
    out_ref[:] = jnp.maximum(out, 0.0)


def kernel(data, W, att_src, att_dst, bias):
    return pl.pallas_call(
        _gat_dense_kernel,
        out_shape=jax.ShapeDtypeStruct((N, OUT_CH), jnp.float32),
    )(
        data,
        W,
        att_src.reshape(1, OUT_CH),
        att_dst.reshape(OUT_CH, 1),
        bias.reshape(1, OUT_CH),
    )
